# baseline (speedup 1.0000x reference)
import functools

import numpy as np
import jax
import jax.numpy as jnp
from jax import lax
from jax.experimental import pallas as pl
from jax.experimental.pallas import tpu as pltpu
from jax.experimental.pallas import tpu_sc as plsc

E = 64
L = 16
N_TOK = 131072
NC, NS = 2, 16
W = NC * NS
TW = N_TOK // W
C = 512
NCHUNK = TW // C
G = C // L

_mesh = plsc.VectorSubcoreMesh(
    core_axis_name="c", subcore_axis_name="s", num_cores=NC, num_subcores=NS
)


@functools.partial(
    pl.kernel,
    out_type=(
        jax.ShapeDtypeStruct((W, E, L), jnp.float32),
        jax.ShapeDtypeStruct((W, E, L), jnp.float32),
    ),
    mesh=_mesh,
    compiler_params=pltpu.CompilerParams(
        needs_layout_passes=False, use_tc_tiling_on_sc=True
    ),
    scratch_types=[
        pltpu.VMEM((E, C), jnp.float32),
        pltpu.VMEM((E, C), jnp.float32),
        pltpu.VMEM((E, L), jnp.float32),
        pltpu.VMEM((E, L), jnp.float32),
        pltpu.SemaphoreType.DMA,
        pltpu.SemaphoreType.DMA,
    ],
)
def _sc_balance(gate_t_hbm, w_out, hit_out, buf0, buf1, accw, acchit,
                sem0, sem1):
    wid = lax.axis_index("s") * NC + lax.axis_index("c")
    tok0 = wid * TW
    zero_v = jnp.zeros((L,), jnp.float32)
    neg_inf = jnp.full((L,), -jnp.inf, dtype=jnp.float32)
    bufs = (buf0, buf1)
    sems = (sem0, sem1)

    def chunk_copy(c_idx, b):
        col = tok0 + jnp.minimum(c_idx, NCHUNK - 1) * C
        col = pl.multiple_of(col, 128)
        return pltpu.make_async_copy(
            gate_t_hbm.at[:, pl.ds(col, C)], bufs[b], sems[b]
        )

    @plsc.parallel_loop(0, E, 1, unroll=8)
    def _init(e):
        accw[e, :] = zero_v
        acchit[e, :] = zero_v

    chunk_copy(0, 0).start()
    chunk_copy(1, 1).start()

    def process(buf):
        def group_body(g, _):
            t = pl.multiple_of(g * L, L)

            @plsc.parallel_loop(
                0, E // 2, 1, unroll=4,
                carry=(zero_v, neg_inf, neg_inf, zero_v, neg_inf, neg_inf),
            )
            def p1(e, carry):
                sa, m1a, m2a, sb, m1b, m2b = carry
                eva = jnp.exp(buf[e, pl.ds(t, L)])
                evb = jnp.exp(buf[e + E // 2, pl.ds(t, L)])
                m2a = jnp.maximum(m2a, jnp.minimum(m1a, eva))
                m1a = jnp.maximum(m1a, eva)
                m2b = jnp.maximum(m2b, jnp.minimum(m1b, evb))
                m1b = jnp.maximum(m1b, evb)
                return sa + eva, m1a, m2a, sb + evb, m1b, m2b

            sa, m1a, m2a, sb, m1b, m2b = p1
            s = sa + sb
            m2 = jnp.maximum(jnp.minimum(m1a, m1b), jnp.maximum(m2a, m2b))
            r = 1.0 / s

            @plsc.parallel_loop(0, E, 1, unroll=8)
            def _p2(e):
                ev = jnp.exp(buf[e, pl.ds(t, L)])
                plsc.addupdate(accw.at[e], ev * r)
                plsc.addupdate(acchit.at[e], jnp.where(ev >= m2, 1.0, 0.0))

            return 0

        lax.fori_loop(0, G, group_body, 0)

    def chunk_body(cc, _):
        for b in range(2):
            c = cc * 2 + b
            chunk_copy(c, b).wait()
            process(bufs[b])
            chunk_copy(c + 2, b).start()
        return 0

    lax.fori_loop(0, NCHUNK // 2, chunk_body, 0)
    chunk_copy(NCHUNK, 0).wait()
    chunk_copy(NCHUNK + 1, 1).wait()

    pltpu.sync_copy(accw, w_out.at[wid])
    pltpu.sync_copy(acchit, hit_out.at[wid])


def kernel(gate_logits):
    w_parts, hit_parts = _sc_balance(gate_logits.T)
    total_w = jnp.sum(w_parts)
    hit_any = jnp.max(hit_parts, axis=(0, 2)) > 0.5
    tpe = jnp.mean(hit_any.astype(jnp.float32))
    return tpe * (total_w / np.float32(N_TOK)) * np.float32(E)


# speedup vs baseline: 1.0116x; 1.0116x over previous
import functools

import numpy as np
import jax
import jax.numpy as jnp
from jax import lax
from jax.experimental import pallas as pl
from jax.experimental.pallas import tpu as pltpu
from jax.experimental.pallas import tpu_sc as plsc

E = 64
L = 16
N_TOK = 131072
NC, NS = 2, 16
W = NC * NS
TW = N_TOK // W
C = 512
NCHUNK = TW // C
G = C // L

_mesh = plsc.VectorSubcoreMesh(
    core_axis_name="c", subcore_axis_name="s", num_cores=NC, num_subcores=NS
)


@functools.partial(
    pl.kernel,
    out_type=(
        jax.ShapeDtypeStruct((W, E, L), jnp.float32),
        jax.ShapeDtypeStruct((W, E, L), jnp.float32),
    ),
    mesh=_mesh,
    compiler_params=pltpu.CompilerParams(
        needs_layout_passes=False, use_tc_tiling_on_sc=True
    ),
    scratch_types=[
        pltpu.VMEM((E, C), jnp.float32),
        pltpu.VMEM((E, C), jnp.float32),
        pltpu.VMEM((E, L), jnp.float32),
        pltpu.VMEM((E, L), jnp.float32),
        pltpu.SemaphoreType.DMA,
        pltpu.SemaphoreType.DMA,
    ],
)
def _sc_balance(gate_t_hbm, w_out, hit_out, buf0, buf1, accw, acchit,
                sem0, sem1):
    wid = lax.axis_index("s") * NC + lax.axis_index("c")
    tok0 = wid * TW
    zero_v = jnp.zeros((L,), jnp.float32)
    neg_inf = jnp.full((L,), -jnp.inf, dtype=jnp.float32)
    bufs = (buf0, buf1)
    sems = (sem0, sem1)

    def chunk_copy(c_idx, b):
        col = tok0 + jnp.minimum(c_idx, NCHUNK - 1) * C
        col = pl.multiple_of(col, 128)
        return pltpu.make_async_copy(
            gate_t_hbm.at[:, pl.ds(col, C)], bufs[b], sems[b]
        )

    @plsc.parallel_loop(0, E, 1, unroll=8)
    def _init(e):
        accw[e, :] = zero_v
        acchit[e, :] = zero_v

    chunk_copy(0, 0).start()
    chunk_copy(1, 1).start()

    def process(buf):
        def group_body(g, _):
            t = pl.multiple_of(g * L, L)

            @plsc.parallel_loop(0, E, 1, unroll=8, carry=(zero_v, neg_inf, neg_inf))
            def p1(e, carry):
                s, m1, m2 = carry
                v = buf[e, pl.ds(t, L)]
                ev = jnp.exp(v)
                m2 = jnp.maximum(m2, jnp.minimum(m1, ev))
                m1 = jnp.maximum(m1, ev)
                return s + ev, m1, m2

            s, _, m2 = p1
            r = 1.0 / s

            @plsc.parallel_loop(0, E, 1, unroll=8)
            def _p2(e):
                ev = jnp.exp(buf[e, pl.ds(t, L)])
                plsc.addupdate(accw.at[e], ev * r)
                plsc.addupdate(acchit.at[e], jnp.where(ev >= m2, 1.0, 0.0))

            return 0

        lax.fori_loop(0, G, group_body, 0)

    def chunk_body(cc, _):
        for b in range(2):
            c = cc * 2 + b
            chunk_copy(c, b).wait()
            process(bufs[b])
            chunk_copy(c + 2, b).start()
        return 0

    lax.fori_loop(0, NCHUNK // 2, chunk_body, 0)
    chunk_copy(NCHUNK, 0).wait()
    chunk_copy(NCHUNK + 1, 1).wait()

    pltpu.sync_copy(accw, w_out.at[wid])
    pltpu.sync_copy(acchit, hit_out.at[wid])


def kernel(gate_logits):
    w_parts, hit_parts = _sc_balance(gate_logits.T)
    total_w = jnp.sum(w_parts)
    hit_any = jnp.max(hit_parts, axis=(0, 2)) > 0.5
    tpe = jnp.mean(hit_any.astype(jnp.float32))
    return tpe * (total_w / np.float32(N_TOK)) * np.float32(E)
